# baseline (speedup 1.0000x reference)
import functools

import jax
import jax.numpy as jnp
from jax import lax
from jax.experimental import pallas as pl
from jax.experimental.pallas import tpu as pltpu
from jax.experimental.pallas import tpu_sc as plsc

EMB = 1024
EPS = 1e-6
LANES = 16
NCHUNK = EMB // LANES

NUM_CORES = 2
NUM_SUBCORES = 16
NW = NUM_CORES * NUM_SUBCORES

R = 64


def _allreduce_sum(x):
    iota = lax.iota(jnp.int32, LANES)
    dnums = lax.GatherDimensionNumbers(
        offset_dims=(), collapsed_slice_dims=(0,), start_index_map=(0,))
    for stride in (1, 2, 4, 8):
        idx = iota ^ stride
        x = x + lax.gather(
            x, idx[:, None], dnums, slice_sizes=(1,),
            mode=lax.GatherScatterMode.PROMISE_IN_BOUNDS)
    return x


def _rsqrt_newton(v):
    i = lax.bitcast_convert_type(v, jnp.int32)
    i = jnp.int32(0x5F3759DF) - (i >> 1)
    y = lax.bitcast_convert_type(i, jnp.float32)
    for _ in range(3):
        y = y * (1.5 - 0.5 * v * y * y)
    return y


def _body(src_ref, table_ref, gamma_ref, beta_ref, out_ref,
          idx_v, rows_v, g_v, b_v, sem):
    wid = lax.axis_index("s") * NUM_CORES + lax.axis_index("c")
    rows_per_w = src_ref.shape[0] // NW
    base = wid * rows_per_w
    n_iters = rows_per_w // R

    pltpu.sync_copy(gamma_ref, g_v)
    pltpu.sync_copy(beta_ref, b_v)

    def chunk(c, carry):
        row0 = base + c * R
        pltpu.sync_copy(src_ref.at[pl.ds(row0, R)], idx_v)
        pltpu.async_copy(table_ref.at[idx_v], rows_v, sem).wait()

        def row(r, rc):
            acc = jnp.zeros((LANES,), jnp.float32)
            acc2 = jnp.zeros((LANES,), jnp.float32)
            for j in range(NCHUNK):
                x = rows_v[r, pl.ds(j * LANES, LANES)]
                acc = acc + x
                acc2 = acc2 + x * x
            s1 = _allreduce_sum(acc)
            s2 = _allreduce_sum(acc2)
            mvec = s1 * (1.0 / EMB)
            var = s2 * (1.0 / EMB) - mvec * mvec
            scale = _rsqrt_newton(var + EPS)
            for j in range(NCHUNK):
                sl = pl.ds(j * LANES, LANES)
                x = rows_v[r, sl]
                a = scale * g_v[sl]
                rows_v[r, sl] = (x - mvec) * a + b_v[sl]
            return rc

        lax.fori_loop(0, R, row, 0)
        pltpu.sync_copy(rows_v, out_ref.at[pl.ds(row0, R)])
        return carry

    lax.fori_loop(0, n_iters, chunk, 0)


def kernel(src, table, gamma, beta):
    b, s = src.shape
    n = b * s
    src_flat = src.reshape(n)
    mesh = plsc.VectorSubcoreMesh(core_axis_name="c", subcore_axis_name="s")
    run = functools.partial(
        pl.kernel,
        mesh=mesh,
        out_type=jax.ShapeDtypeStruct((n, EMB), jnp.float32),
        scratch_types=[
            pltpu.VMEM((R,), jnp.int32),
            pltpu.VMEM((R, EMB), jnp.float32),
            pltpu.VMEM((EMB,), jnp.float32),
            pltpu.VMEM((EMB,), jnp.float32),
            pltpu.SemaphoreType.DMA,
        ],
    )(_body)
    out = run(src_flat, table, gamma, beta)
    return out.reshape(b, s, EMB)
